# baseline (speedup 1.0000x reference)
import functools

import jax
import jax.numpy as jnp
from jax import lax
from jax.experimental import pallas as pl
from jax.experimental.pallas import tpu as pltpu
from jax.experimental.pallas import tpu_sc as plsc

ROWS = 1000000
DIM = 32
NH = 4
B = 16384
P = 2147483647
M15 = 0x7FFF
M16 = 0xFFFF

NC = 2
NS = 16
L = 16
NW = NC * NS
BW = B // NW
NCHUNK = 16
CB = BW // NCHUNK
CIDX = CB * NH


def _addmod(u, v):
    s = u + v
    return jnp.where(s < 0, (s & P) + 1, s)


def _hash16(xv, a1, a0, bv):
    x1 = xv >> 15
    x0 = xv & M15
    v = a1 * x1
    t30 = (v >> 1) + ((v & 1) << 30)
    v = a1 * x0
    t15a = (v >> 16) + ((v & M16) << 15)
    v = a0 * x1
    t15b = (v >> 16) + ((v & M16) << 15)
    t0 = a0 * x0
    s = _addmod(_addmod(t30, t15a), _addmod(t15b, t0))
    s = _addmod(s, bv)
    h = jnp.where(s == P, 0, s)
    for k in range(11, -1, -1):
        c = ROWS << k
        h = jnp.where(h >= c, h - c, h)
    return h


def _body(x_hbm, ab_hbm, table_hbm, out_hbm, x_v, ab_v, idx_v, rows_v, out_v,
          sem, outsem):
    wid = lax.axis_index("s") * NC + lax.axis_index("c")
    base = wid * BW

    pltpu.sync_copy(x_hbm.at[pl.ds(base, BW)], x_v)
    pltpu.sync_copy(ab_hbm, ab_v)

    A1 = [ab_v[h] >> 15 for h in range(NH)]
    A0 = [ab_v[h] & M15 for h in range(NH)]
    BV = [ab_v[NH + h] for h in range(NH)]

    def hash_chunk(s, _):
        s = s.astype(jnp.int32)
        for g in range(CB // L):
            xv = x_v[pl.ds(s * CB + g * L, L)]
            for h in range(NH):
                idx_v[s, pl.ds(h * CB + g * L, L)] = _hash16(xv, A1[h], A0[h],
                                                             BV[h])
        pltpu.async_copy(table_hbm.at[idx_v.at[s]], rows_v.at[s], sem)
        return 0

    lax.fori_loop(jnp.int32(0), jnp.int32(NCHUNK), hash_chunk, 0,
                  unroll=False)

    z = jnp.int32(0)
    for _ in range(NCHUNK):
        pltpu.make_async_copy(table_hbm.at[idx_v.at[z]], rows_v.at[z],
                              sem).wait()

    def sum_chunk(s, _):
        s = s.astype(jnp.int32)
        for u in range(CB):
            e = s * CB + u
            acc = rows_v[s, 0 * CB + u, :] + rows_v[s, 1 * CB + u, :]
            acc = acc + (rows_v[s, 2 * CB + u, :] + rows_v[s, 3 * CB + u, :])
            out_v[e, :] = acc * jnp.bfloat16(0.25)
        return 0

    lax.fori_loop(jnp.int32(0), jnp.int32(NCHUNK), sum_chunk, 0,
                  unroll=False)

    pltpu.async_copy(out_v, out_hbm.at[pl.ds(base, BW), :], outsem).wait()


@jax.jit
def _bloom(x32, ab, table16):
    mesh = plsc.VectorSubcoreMesh(core_axis_name="c", subcore_axis_name="s")
    f = functools.partial(
        pl.kernel,
        mesh=mesh,
        out_type=jax.ShapeDtypeStruct((B, DIM), jnp.bfloat16),
        scratch_types=[
            pltpu.VMEM((BW,), jnp.int32),
            pltpu.VMEM((2 * NH, L), jnp.int32),
            pltpu.VMEM((NCHUNK, CIDX), jnp.int32),
            pltpu.VMEM((NCHUNK, CIDX, DIM), jnp.bfloat16),
            pltpu.VMEM((BW, DIM), jnp.bfloat16),
            pltpu.SemaphoreType.DMA,
            pltpu.SemaphoreType.DMA,
        ],
        compiler_params=pltpu.CompilerParams(use_tc_tiling_on_sc=False),
    )(_body)
    return f(x32, ab, table16)


def kernel(x, table, a, b):
    x32 = x.astype(jnp.int32)
    ab = jnp.concatenate([a.astype(jnp.int32), b.astype(jnp.int32)])
    ab = jnp.broadcast_to(ab[:, None], (2 * NH, L))
    out = _bloom(x32, ab, table.astype(jnp.bfloat16))
    return out.astype(jnp.float32)


# speedup vs baseline: 1.7232x; 1.7232x over previous
import functools

import jax
import jax.numpy as jnp
from jax import lax
from jax.experimental import pallas as pl
from jax.experimental.pallas import tpu as pltpu
from jax.experimental.pallas import tpu_sc as plsc

ROWS = 1000000
DIM = 32
NH = 4
B = 16384
P = 2147483647
M15 = 0x7FFF
M16 = 0xFFFF

NC = 2
NS = 16
L = 16
NW = NC * NS
BW = B // NW
NCHUNK = 8
CB = BW // NCHUNK
CIDX = CB * NH
NVEC = CIDX // L


def _addmod(u, v):
    s = u + v
    return jnp.where(s < 0, (s & P) + 1, s)


def _hash16(xv, a1, a0, bv):
    x1 = xv >> 15
    x0 = xv & M15
    v = a1 * x1
    t30 = (v >> 1) + ((v & 1) << 30)
    v = a1 * x0
    t15a = (v >> 16) + ((v & M16) << 15)
    v = a0 * x1
    t15b = (v >> 16) + ((v & M16) << 15)
    t0 = a0 * x0
    s = _addmod(_addmod(t30, t15a), _addmod(t15b, t0))
    s = _addmod(s, bv)
    h = jnp.where(s == P, 0, s)
    for k in range(11, -1, -1):
        c = ROWS << k
        h = jnp.where(h >= c, h - c, h)
    return h


def _body(x_hbm, ab_hbm, table_hbm, out_hbm, x_v, ab_v, rows_v, out_v, sem,
          outsem):
    wid = lax.axis_index("s") * NC + lax.axis_index("c")
    base = wid * BW

    pltpu.sync_copy(x_hbm.at[pl.ds(base, BW)], x_v)
    pltpu.sync_copy(ab_hbm, ab_v)

    A1 = [ab_v[h] >> 15 for h in range(NH)]
    A0 = [ab_v[h] & M15 for h in range(NH)]
    BV = [ab_v[NH + h] for h in range(NH)]

    def hash_chunk(s):
        out = []
        for g in range(CB // L):
            xv = x_v[pl.ds(s * CB + g * L, L)]
            for h in range(NH):
                out.append(_hash16(xv, A1[h], A0[h], BV[h]))
        return out

    def slot(vi, j):
        g, h = divmod(vi, NH)
        return h * CB + g * L + j

    def chunk_step(s, carry):
        s = s.astype(jnp.int32)
        for vi in range(NVEC):
            v = carry[vi]
            for j in range(L):
                r = v[j]
                pltpu.async_copy(table_hbm.at[pl.ds(r, 1), :],
                                 rows_v.at[pl.ds(slot(vi, j), 1), :], sem)
        nxt = hash_chunk(jnp.minimum(s + 1, NCHUNK - 1))
        z = jnp.int32(0)
        for _ in range(CIDX):
            pltpu.make_async_copy(table_hbm.at[pl.ds(z, 1), :],
                                  rows_v.at[pl.ds(z, 1), :], sem).wait()
        for u in range(CB):
            e = s * CB + u
            for half in range(DIM // L):
                acc = (rows_v[0 * CB + u, pl.ds(half * L, L)]
                       + rows_v[1 * CB + u, pl.ds(half * L, L)])
                acc = acc + (rows_v[2 * CB + u, pl.ds(half * L, L)]
                             + rows_v[3 * CB + u, pl.ds(half * L, L)])
                out_v[e, pl.ds(half * L, L)] = acc * 0.25
        return nxt

    lax.fori_loop(jnp.int32(0), jnp.int32(NCHUNK), chunk_step,
                  hash_chunk(jnp.int32(0)), unroll=False)

    pltpu.async_copy(out_v, out_hbm.at[pl.ds(base, BW), :], outsem).wait()


@jax.jit
def _bloom(x32, ab, table):
    mesh = plsc.VectorSubcoreMesh(core_axis_name="c", subcore_axis_name="s")
    f = functools.partial(
        pl.kernel,
        mesh=mesh,
        out_type=jax.ShapeDtypeStruct((B, DIM), jnp.float32),
        scratch_types=[
            pltpu.VMEM((BW,), jnp.int32),
            pltpu.VMEM((2 * NH, L), jnp.int32),
            pltpu.VMEM((CIDX, DIM), jnp.float32),
            pltpu.VMEM((BW, DIM), jnp.float32),
            pltpu.SemaphoreType.DMA,
            pltpu.SemaphoreType.DMA,
        ],
        compiler_params=pltpu.CompilerParams(use_tc_tiling_on_sc=True,
                                             needs_layout_passes=False),
    )(_body)
    return f(x32, ab, table)


def kernel(x, table, a, b):
    x32 = x.astype(jnp.int32)
    ab = jnp.concatenate([a.astype(jnp.int32), b.astype(jnp.int32)])
    ab = jnp.broadcast_to(ab[:, None], (2 * NH, L))
    return _bloom(x32, ab, table)


# speedup vs baseline: 1.7792x; 1.0325x over previous
import functools

import jax
import jax.numpy as jnp
from jax import lax
from jax.experimental import pallas as pl
from jax.experimental.pallas import tpu as pltpu
from jax.experimental.pallas import tpu_sc as plsc

ROWS = 1000000
DIM = 32
NH = 4
B = 16384
P = 2147483647
M15 = 0x7FFF
M16 = 0xFFFF

NC = 2
NS = 16
L = 16
NW = NC * NS
BW = B // NW
NCHUNK = 16
CB = BW // NCHUNK
CIDX = CB * NH
NVEC = CIDX // L


def _addmod(u, v):
    s = u + v
    return jnp.where(s < 0, (s & P) + 1, s)


def _hash16(xv, a1, a0, bv):
    x1 = xv >> 15
    x0 = xv & M15
    v = a1 * x1
    t30 = (v >> 1) + ((v & 1) << 30)
    v = a1 * x0
    t15a = (v >> 16) + ((v & M16) << 15)
    v = a0 * x1
    t15b = (v >> 16) + ((v & M16) << 15)
    t0 = a0 * x0
    s = _addmod(_addmod(t30, t15a), _addmod(t15b, t0))
    s = _addmod(s, bv)
    h = jnp.where(s == P, 0, s)
    for k in range(11, -1, -1):
        c = ROWS << k
        h = jnp.where(h >= c, h - c, h)
    return h


def _body(x_hbm, ab_hbm, table_hbm, out_hbm, x_v, ab_v, rows_v, out_v, sem,
          outsem):
    wid = lax.axis_index("s") * NC + lax.axis_index("c")
    base = wid * BW

    pltpu.sync_copy(x_hbm.at[pl.ds(base, BW)], x_v)
    pltpu.sync_copy(ab_hbm, ab_v)

    A1 = [ab_v[h] >> 15 for h in range(NH)]
    A0 = [ab_v[h] & M15 for h in range(NH)]
    BV = [ab_v[NH + h] for h in range(NH)]

    def hash_chunk(s):
        out = []
        for g in range(CB // L):
            xv = x_v[pl.ds(s * CB + g * L, L)]
            for h in range(NH):
                out.append(_hash16(xv, A1[h], A0[h], BV[h]))
        return out

    def slot(vi, j):
        g, h = divmod(vi, NH)
        return h * CB + g * L + j

    def chunk_step(s, carry):
        s = s.astype(jnp.int32)
        for vi in range(NVEC):
            v = carry[vi]
            for j in range(L):
                r = v[j]
                pltpu.async_copy(table_hbm.at[pl.ds(r, 1), :],
                                 rows_v.at[pl.ds(slot(vi, j), 1), :], sem)
        nxt = hash_chunk(jnp.minimum(s + 1, NCHUNK - 1))
        z = jnp.int32(0)
        for _ in range(CIDX):
            pltpu.make_async_copy(table_hbm.at[pl.ds(z, 1), :],
                                  rows_v.at[pl.ds(z, 1), :], sem).wait()
        for u in range(CB):
            e = s * CB + u
            for half in range(DIM // L):
                acc = (rows_v[0 * CB + u, pl.ds(half * L, L)]
                       + rows_v[1 * CB + u, pl.ds(half * L, L)])
                acc = acc + (rows_v[2 * CB + u, pl.ds(half * L, L)]
                             + rows_v[3 * CB + u, pl.ds(half * L, L)])
                out_v[e, pl.ds(half * L, L)] = acc * 0.25
        return nxt

    lax.fori_loop(jnp.int32(0), jnp.int32(NCHUNK), chunk_step,
                  hash_chunk(jnp.int32(0)), unroll=False)

    pltpu.async_copy(out_v, out_hbm.at[pl.ds(base, BW), :], outsem).wait()


@jax.jit
def _bloom(x32, ab, table):
    mesh = plsc.VectorSubcoreMesh(core_axis_name="c", subcore_axis_name="s")
    f = functools.partial(
        pl.kernel,
        mesh=mesh,
        out_type=jax.ShapeDtypeStruct((B, DIM), jnp.float32),
        scratch_types=[
            pltpu.VMEM((BW,), jnp.int32),
            pltpu.VMEM((2 * NH, L), jnp.int32),
            pltpu.VMEM((CIDX, DIM), jnp.float32),
            pltpu.VMEM((BW, DIM), jnp.float32),
            pltpu.SemaphoreType.DMA,
            pltpu.SemaphoreType.DMA,
        ],
        compiler_params=pltpu.CompilerParams(use_tc_tiling_on_sc=True,
                                             needs_layout_passes=False),
    )(_body)
    return f(x32, ab, table)


def kernel(x, table, a, b):
    x32 = x.astype(jnp.int32)
    ab = jnp.concatenate([a.astype(jnp.int32), b.astype(jnp.int32)])
    ab = jnp.broadcast_to(ab[:, None], (2 * NH, L))
    return _bloom(x32, ab, table)
